# baseline (speedup 1.0000x reference)
import numpy as np
import jax
import jax.numpy as jnp
from jax.experimental import pallas as pl
from jax.experimental.pallas import tpu as pltpu


def _fold_bn(gamma, beta, mean, var, eps=1e-5):
    scale = gamma / jnp.sqrt(var + eps)
    bias = beta - mean * scale
    return scale, bias


def _band_blocks(w_hwio, c):
    bw = 256 // c
    dw = bw + 2
    sel = np.zeros((3, dw, bw), np.float32)
    for kx in range(3):
        for o in range(bw):
            sel[kx, o + kx, o] = 1.0
    b = jnp.einsum("xdo,yxic->dyioc", jnp.asarray(sel),
                   w_hwio.astype(jnp.float32))
    return b.reshape(dw * 3 * c, bw * c).astype(jnp.bfloat16)


def _bb_kernel(x_ref, b1_ref, s1_ref, t1_ref, b2_ref, s2_ref, t2_ref,
               out_ref):
    bt, H, WC = x_ref.shape
    nblk = WC // 256
    xf = x_ref[...]
    x = xf.astype(jnp.bfloat16)

    def conv_bn(padded, b_ref, s_ref, t_ref):
        up = jnp.roll(padded, 1, axis=1)
        dn = jnp.roll(padded, H - 1, axis=1)
        rows = bt * H
        ng = (WC + 64) // 32
        t2 = jnp.concatenate(
            [up.reshape(bt, H, ng, 32), padded.reshape(bt, H, ng, 32),
             dn.reshape(bt, H, ng, 32)], axis=-1).reshape(rows, ng * 96)
        outs = []
        for j in range(nblk):
            acc = jnp.dot(t2[:, 768 * j:768 * j + 960], b_ref[...],
                          preferred_element_type=jnp.float32)
            lo = 256 * j
            outs.append(acc * s_ref[:, lo:lo + 256] + t_ref[:, lo:lo + 256])
        return outs

    xp = jnp.concatenate([x[..., WC - 32:], x, x[..., :32]], axis=-1)
    h1 = [jnp.maximum(o, 0.0).astype(jnp.bfloat16).reshape(bt, H, 256)
          for o in conv_bn(xp, b1_ref, s1_ref, t1_ref)]
    h1p = jnp.concatenate([h1[-1][..., 224:]] + h1 + [h1[0][..., :32]],
                          axis=-1)
    h2 = conv_bn(h1p, b2_ref, s2_ref, t2_ref)
    res = jnp.concatenate(h2, axis=-1).reshape(bt, H, WC) + xf
    out_ref[...] = jnp.maximum(res, 0.0).astype(out_ref.dtype)


def kernel(x_nchw, w1, w2, bn1_gamma, bn1_beta, bn1_mean, bn1_var,
           bn2_gamma, bn2_beta, bn2_mean, bn2_var):
    N, C, H, W = x_nchw.shape
    WC = W * C

    x = jnp.transpose(x_nchw, (0, 2, 3, 1)).reshape(N, H, WC)

    s1, b1 = _fold_bn(bn1_gamma, bn1_beta, bn1_mean, bn1_var)
    s2, b2 = _fold_bn(bn2_gamma, bn2_beta, bn2_mean, bn2_var)
    s1r = jnp.tile(s1, W)[None, :].astype(jnp.float32)
    t1r = jnp.tile(b1, W)[None, :].astype(jnp.float32)
    s2r = jnp.tile(s2, W)[None, :].astype(jnp.float32)
    t2r = jnp.tile(b2, W)[None, :].astype(jnp.float32)

    bb1 = _band_blocks(w1, C)
    bb2 = _band_blocks(w2, C)

    bt = next(d for d in (32, 16, 8, 4, 2, 1) if N % d == 0)
    grid = (N // bt,)

    const = lambda n: (0, 0)
    out = pl.pallas_call(
        _bb_kernel,
        out_shape=jax.ShapeDtypeStruct((N, H, WC), jnp.bfloat16),
        grid=grid,
        in_specs=[
            pl.BlockSpec((bt, H, WC), lambda n: (n, 0, 0)),
            pl.BlockSpec((960, 256), const),
            pl.BlockSpec((1, WC), const),
            pl.BlockSpec((1, WC), const),
            pl.BlockSpec((960, 256), const),
            pl.BlockSpec((1, WC), const),
            pl.BlockSpec((1, WC), const),
        ],
        out_specs=pl.BlockSpec((bt, H, WC), lambda n: (n, 0, 0)),
        compiler_params=pltpu.CompilerParams(
            dimension_semantics=("parallel",)),
    )(x, bb1, s1r, t1r, bb2, s2r, t2r)

    return jnp.transpose(out.reshape(N, H, W, C),
                         (0, 3, 1, 2)).astype(x_nchw.dtype)


# speedup vs baseline: 2.3989x; 2.3989x over previous
import numpy as np
import jax
import jax.numpy as jnp
from jax.experimental import pallas as pl
from jax.experimental.pallas import tpu as pltpu


def _fold_bn(gamma, beta, mean, var, eps=1e-5):
    scale = gamma / jnp.sqrt(var + eps)
    bias = beta - mean * scale
    return scale, bias


def _band_blocks(w_hwio, c):
    bw = 256 // c
    dw = bw + 2
    sel = np.zeros((3, dw, bw), np.float32)
    for kx in range(3):
        for o in range(bw):
            sel[kx, o + kx, o] = 1.0
    b = jnp.einsum("xdo,yxic->ydioc", jnp.asarray(sel),
                   w_hwio.astype(jnp.float32))
    return b.reshape(3 * dw * c, bw * c).astype(jnp.bfloat16)


def _bb_kernel(x_ref, b1_ref, s1_ref, t1_ref, b2_ref, s2_ref, t2_ref,
               out_ref):
    bt, H, WC = x_ref.shape
    nblk = WC // 256
    xf = x_ref[...]
    x = xf.astype(jnp.bfloat16)

    def conv_bn(padded, b_ref, s_ref, t_ref):
        up = jnp.roll(padded, 1, axis=1)
        dn = jnp.roll(padded, H - 1, axis=1)
        rows = bt * H
        a = padded.reshape(rows, WC + 64)
        u = up.reshape(rows, WC + 64)
        d = dn.reshape(rows, WC + 64)
        outs = []
        for j in range(nblk):
            lo = 256 * j
            lhs = jnp.concatenate(
                [u[:, lo:lo + 320], a[:, lo:lo + 320], d[:, lo:lo + 320]],
                axis=1)
            acc = jnp.dot(lhs, b_ref[...],
                          preferred_element_type=jnp.float32)
            outs.append(acc * s_ref[:, lo:lo + 256] + t_ref[:, lo:lo + 256])
        return outs

    xp = jnp.concatenate([x[..., WC - 32:], x, x[..., :32]], axis=-1)
    h1 = [jnp.maximum(o, 0.0).astype(jnp.bfloat16).reshape(bt, H, 256)
          for o in conv_bn(xp, b1_ref, s1_ref, t1_ref)]
    h1p = jnp.concatenate([h1[-1][..., 224:]] + h1 + [h1[0][..., :32]],
                          axis=-1)
    h2 = conv_bn(h1p, b2_ref, s2_ref, t2_ref)
    res = jnp.concatenate(h2, axis=-1).reshape(bt, H, WC) + xf
    out_ref[...] = jnp.maximum(res, 0.0).astype(out_ref.dtype)


def kernel(x_nchw, w1, w2, bn1_gamma, bn1_beta, bn1_mean, bn1_var,
           bn2_gamma, bn2_beta, bn2_mean, bn2_var):
    N, C, H, W = x_nchw.shape
    WC = W * C

    x = jnp.transpose(x_nchw, (0, 2, 3, 1)).reshape(N, H, WC)

    s1, b1 = _fold_bn(bn1_gamma, bn1_beta, bn1_mean, bn1_var)
    s2, b2 = _fold_bn(bn2_gamma, bn2_beta, bn2_mean, bn2_var)
    s1r = jnp.tile(s1, W)[None, :].astype(jnp.float32)
    t1r = jnp.tile(b1, W)[None, :].astype(jnp.float32)
    s2r = jnp.tile(s2, W)[None, :].astype(jnp.float32)
    t2r = jnp.tile(b2, W)[None, :].astype(jnp.float32)

    bb1 = _band_blocks(w1, C)
    bb2 = _band_blocks(w2, C)

    bt = next(d for d in (32, 16, 8, 4, 2, 1) if N % d == 0)
    grid = (N // bt,)

    const = lambda n: (0, 0)
    out = pl.pallas_call(
        _bb_kernel,
        out_shape=jax.ShapeDtypeStruct((N, H, WC), jnp.bfloat16),
        grid=grid,
        in_specs=[
            pl.BlockSpec((bt, H, WC), lambda n: (n, 0, 0)),
            pl.BlockSpec((960, 256), const),
            pl.BlockSpec((1, WC), const),
            pl.BlockSpec((1, WC), const),
            pl.BlockSpec((960, 256), const),
            pl.BlockSpec((1, WC), const),
            pl.BlockSpec((1, WC), const),
        ],
        out_specs=pl.BlockSpec((bt, H, WC), lambda n: (n, 0, 0)),
        compiler_params=pltpu.CompilerParams(
            dimension_semantics=("parallel",)),
    )(x, bb1, s1r, t1r, bb2, s2r, t2r)

    return jnp.transpose(out.reshape(N, H, W, C),
                         (0, 3, 1, 2)).astype(x_nchw.dtype)
